# baseline (device time: 56661 ns/iter reference)
import jax
import jax.numpy as jnp
from jax import lax
from jax.experimental import pallas as pl
from jax.experimental.pallas import tpu as pltpu

N_DEV = 4


def kernel(ids, E):
    T = ids.shape[0]
    V_per, D = E.shape
    H = T // 2
    Q = H // 4

    my = lax.axis_index("i")
    x0 = my // 2
    y0 = lax.rem((my + 1) // 2, 2)

    loc = ids - my * V_per
    mask = (loc >= 0) & (loc < V_per)
    safe = jnp.where(mask, loc, 0).astype(jnp.int32)
    maskf = mask.astype(jnp.bfloat16)[:, None]

    t_idx = jnp.arange(T, dtype=jnp.int32)
    blk = t_idx // (2 * Q)
    is_keep = jnp.where(t_idx < H, blk == x0, blk - 2 == y0)
    piece = lax.rem(t_idx // Q, 2)
    is_b = (t_idx >= H).astype(jnp.int32)
    seg = jnp.where(is_keep, 4, 0) + piece * 2 + is_b
    key = jnp.where(mask, seg, 8)
    packed = lax.sort(key * (1 << 25) + safe * (1 << 11) + t_idx,
                      is_stable=False)
    cum = jnp.cumsum(
        jnp.sum(jnp.where(key[None, :] == jnp.arange(8)[:, None], 1, 0), axis=1)
    ).astype(jnp.int32)

    def body(packed_ref, cum_ref, maskf_ref, e_ref, out_ref, gbuf, red_ref,
             ostage, rs1_buf, rs2_buf, gsem, osem, p_send, p_recv):
        my_pos = lax.axis_index("i")
        xr = my_pos // 2
        yr = lax.rem((my_pos + 1) // 2, 2)
        xp = 3 - my_pos
        yp = my_pos + 1 - 2 * lax.rem(my_pos, 2)

        a_send = (1 - xr) * 2 * Q
        a_keep = xr * 2 * Q
        b_send = H + (1 - yr) * 2 * Q
        b_keep = H + yr * 2 * Q
        keep = (a_keep, b_keep)
        send = (a_send, b_send)
        partners = ((xp, yp), (yp, xp), (xp, yp))

        def issue_seg(s, lo, hi):
            def fn(t, _):
                v = packed_ref[t]
                pltpu.make_async_copy(
                    e_ref.at[(v >> 11) & (16 * 1024 - 1)],
                    gbuf.at[v & (2 * 1024 - 1)],
                    gsem.at[s],
                ).start()
                return 0

            lax.fori_loop(lo, hi, fn, 0)

        def drain_convert(s, lo, hi, start):
            def fn(t, _):
                pltpu.make_async_copy(
                    e_ref.at[0], gbuf.at[0], gsem.at[s]
                ).wait()
                return 0

            lax.fori_loop(0, hi - lo, fn, 0)
            sl = pl.ds(start, Q)
            red_ref[sl] = jnp.where(
                maskf_ref[sl] != 0, gbuf[sl].astype(jnp.bfloat16),
                jnp.bfloat16(0),
            )

        def start_piece(ph, half, pc, src_row, dst):
            rdma = pltpu.make_async_remote_copy(
                src_ref=red_ref.at[pl.ds(src_row, Q)],
                dst_ref=dst,
                send_sem=p_send.at[ph, half, pc],
                recv_sem=p_recv.at[ph, half, pc],
                device_id=(partners[ph][half],),
                device_id_type=pl.DeviceIdType.MESH,
            )
            rdma.start()
            return rdma

        def accum(start, buf):
            sl = pl.ds(start, Q)
            red_ref[sl] = red_ref[sl] + buf

        def oflush(start):
            sl = pl.ds(start, Q)
            ostage[sl] = red_ref[sl].astype(jnp.float32)
            pltpu.make_async_copy(ostage.at[sl], out_ref.at[sl], osem).start()

        for s in range(4):
            issue_seg(s, 0 if s == 0 else cum_ref[s - 1], cum_ref[s])

        barrier_sem = pltpu.get_barrier_semaphore()
        for nbr in (xp, yp):
            pl.semaphore_signal(
                barrier_sem, inc=1,
                device_id=(nbr,), device_id_type=pl.DeviceIdType.MESH,
            )
        pl.semaphore_wait(barrier_sem, 2)

        p1 = {}
        for s, (half, pc) in enumerate(((0, 0), (1, 0), (0, 1), (1, 1))):
            drain_convert(s, 0 if s == 0 else cum_ref[s - 1],
                          cum_ref[s], send[half] + pc * Q)
            p1[half, pc] = start_piece(0, half, pc, send[half] + pc * Q,
                                       rs1_buf.at[half, pc])

        for s in range(4, 8):
            issue_seg(s, cum_ref[s - 1], cum_ref[s])
        for s, (half, pc) in enumerate(((0, 0), (1, 0), (0, 1), (1, 1))):
            drain_convert(s + 4, cum_ref[s + 3], cum_ref[s + 4],
                          keep[half] + pc * Q)

        p2 = {}
        for pc in range(2):
            for half in range(2):
                p1[half, pc].wait()
                accum(keep[half] + pc * Q, rs1_buf[half, pc])
                p2[half, pc] = start_piece(1, half, pc, keep[half] + pc * Q,
                                           rs2_buf.at[half, pc])

        p3 = {}
        for pc in range(2):
            for half in range(2):
                p2[half, pc].wait()
                accum(keep[half] + pc * Q, rs2_buf[half, pc])
                p3[half, pc] = start_piece(
                    2, half, pc, keep[half] + pc * Q,
                    red_ref.at[pl.ds(keep[half] + pc * Q, Q)],
                )
                oflush(keep[half] + pc * Q)

        for pc in range(2):
            for half in range(2):
                p3[half, pc].wait()
                oflush(send[half] + pc * Q)

        for _ in range(8):
            pltpu.make_async_copy(
                ostage.at[pl.ds(0, Q)], out_ref.at[pl.ds(0, Q)], osem
            ).wait()

    return pl.pallas_call(
        body,
        out_shape=jax.ShapeDtypeStruct((T, D), jnp.float32),
        in_specs=[
            pl.BlockSpec(memory_space=pltpu.SMEM),
            pl.BlockSpec(memory_space=pltpu.SMEM),
            pl.BlockSpec(memory_space=pltpu.VMEM),
            pl.BlockSpec(memory_space=pl.ANY),
        ],
        out_specs=pl.BlockSpec(memory_space=pl.ANY),
        scratch_shapes=[
            pltpu.VMEM((T, D), jnp.float32),
            pltpu.VMEM((T, D), jnp.bfloat16),
            pltpu.VMEM((T, D), jnp.float32),
            pltpu.VMEM((2, 2, Q, D), jnp.bfloat16),
            pltpu.VMEM((2, 2, Q, D), jnp.bfloat16),
            pltpu.SemaphoreType.DMA((8,)),
            pltpu.SemaphoreType.DMA,
            pltpu.SemaphoreType.DMA((3, 2, 2)),
            pltpu.SemaphoreType.DMA((3, 2, 2)),
        ],
        compiler_params=pltpu.CompilerParams(collective_id=0),
    )(packed, cum, maskf, E)


# device time: 54178 ns/iter; 1.0458x vs baseline; 1.0458x over previous
import jax
import jax.numpy as jnp
from jax import lax
from jax.experimental import pallas as pl
from jax.experimental.pallas import tpu as pltpu

N_DEV = 4


def kernel(ids, E):
    T = ids.shape[0]
    V_per, D = E.shape
    H = T // 2
    Q = H // 4

    my = lax.axis_index("i")
    x0 = my // 2
    y0 = lax.rem((my + 1) // 2, 2)

    loc = ids - my * V_per
    mask = (loc >= 0) & (loc < V_per)
    safe = jnp.where(mask, loc, 0).astype(jnp.int32)
    maskf = mask.astype(jnp.bfloat16)[:, None]

    t_idx = jnp.arange(T, dtype=jnp.int32)
    blk = t_idx // (2 * Q)
    is_keep = jnp.where(t_idx < H, blk == x0, blk - 2 == y0)
    piece = lax.rem(t_idx // Q, 2)
    is_b = (t_idx >= H).astype(jnp.int32)
    seg = jnp.where(is_keep, 4, 0) + piece * 2 + is_b
    key = jnp.where(mask, seg, 8)
    packed = lax.sort(key * (1 << 25) + safe * (1 << 11) + t_idx,
                      is_stable=False)
    cum = jnp.cumsum(
        jnp.sum(jnp.where(key[None, :] == jnp.arange(8)[:, None], 1, 0), axis=1)
    ).astype(jnp.int32)

    def body(packed_ref, cum_ref, maskf_ref, e_ref, out_ref, gbuf, red_ref,
             rs1_buf, rs2_buf, gsem, osem, p_send, p_recv):
        my_pos = lax.axis_index("i")
        xr = my_pos // 2
        yr = lax.rem((my_pos + 1) // 2, 2)
        xp = 3 - my_pos
        yp = my_pos + 1 - 2 * lax.rem(my_pos, 2)

        a_send = (1 - xr) * 2 * Q
        a_keep = xr * 2 * Q
        b_send = H + (1 - yr) * 2 * Q
        b_keep = H + yr * 2 * Q
        keep = (a_keep, b_keep)
        send = (a_send, b_send)
        partners = ((xp, yp), (yp, xp), (xp, yp))

        def issue_seg(s, lo, hi):
            def fn(t, _):
                v = packed_ref[t]
                pltpu.make_async_copy(
                    e_ref.at[(v >> 11) & (16 * 1024 - 1)],
                    gbuf.at[v & (2 * 1024 - 1)],
                    gsem.at[s],
                ).start()
                return 0

            lax.fori_loop(lo, hi, fn, 0)

        def drain_convert(s, lo, hi, start):
            def fn(t, _):
                pltpu.make_async_copy(
                    e_ref.at[0], gbuf.at[0], gsem.at[s]
                ).wait()
                return 0

            lax.fori_loop(0, hi - lo, fn, 0)
            sl = pl.ds(start, Q)
            red_ref[sl] = jnp.where(
                maskf_ref[sl] != 0, gbuf[sl].astype(jnp.bfloat16),
                jnp.bfloat16(0),
            )

        def start_piece(ph, half, pc, src_row, dst):
            rdma = pltpu.make_async_remote_copy(
                src_ref=red_ref.at[pl.ds(src_row, Q)],
                dst_ref=dst,
                send_sem=p_send.at[ph, half, pc],
                recv_sem=p_recv.at[ph, half, pc],
                device_id=(partners[ph][half],),
                device_id_type=pl.DeviceIdType.MESH,
            )
            rdma.start()
            return rdma

        def accum(start, buf):
            sl = pl.ds(start, Q)
            red_ref[sl] = red_ref[sl] + buf

        def oflush(start):
            sl = pl.ds(start, Q)
            pltpu.make_async_copy(red_ref.at[sl], out_ref.at[sl], osem).start()

        for s in range(4):
            issue_seg(s, 0 if s == 0 else cum_ref[s - 1], cum_ref[s])

        barrier_sem = pltpu.get_barrier_semaphore()
        for nbr in (xp, yp):
            pl.semaphore_signal(
                barrier_sem, inc=1,
                device_id=(nbr,), device_id_type=pl.DeviceIdType.MESH,
            )
        pl.semaphore_wait(barrier_sem, 2)

        p1 = {}
        for s, (half, pc) in enumerate(((0, 0), (1, 0), (0, 1), (1, 1))):
            drain_convert(s, 0 if s == 0 else cum_ref[s - 1],
                          cum_ref[s], send[half] + pc * Q)
            p1[half, pc] = start_piece(0, half, pc, send[half] + pc * Q,
                                       rs1_buf.at[half, pc])

        for s in range(4, 8):
            issue_seg(s, cum_ref[s - 1], cum_ref[s])
        for s, (half, pc) in enumerate(((0, 0), (1, 0), (0, 1), (1, 1))):
            drain_convert(s + 4, cum_ref[s + 3], cum_ref[s + 4],
                          keep[half] + pc * Q)

        p2 = {}
        for pc in range(2):
            for half in range(2):
                p1[half, pc].wait()
                accum(keep[half] + pc * Q, rs1_buf[half, pc])
                p2[half, pc] = start_piece(1, half, pc, keep[half] + pc * Q,
                                           rs2_buf.at[half, pc])

        p3 = {}
        for pc in range(2):
            for half in range(2):
                p2[half, pc].wait()
                accum(keep[half] + pc * Q, rs2_buf[half, pc])
                p3[half, pc] = start_piece(
                    2, half, pc, keep[half] + pc * Q,
                    red_ref.at[pl.ds(keep[half] + pc * Q, Q)],
                )
                oflush(keep[half] + pc * Q)

        for pc in range(2):
            for half in range(2):
                p3[half, pc].wait()
                oflush(send[half] + pc * Q)

        for _ in range(8):
            pltpu.make_async_copy(
                red_ref.at[pl.ds(0, Q)], out_ref.at[pl.ds(0, Q)], osem
            ).wait()

    return pl.pallas_call(
        body,
        out_shape=jax.ShapeDtypeStruct((T, D), jnp.bfloat16),
        in_specs=[
            pl.BlockSpec(memory_space=pltpu.SMEM),
            pl.BlockSpec(memory_space=pltpu.SMEM),
            pl.BlockSpec(memory_space=pltpu.VMEM),
            pl.BlockSpec(memory_space=pl.ANY),
        ],
        out_specs=pl.BlockSpec(memory_space=pl.ANY),
        scratch_shapes=[
            pltpu.VMEM((T, D), jnp.float32),
            pltpu.VMEM((T, D), jnp.bfloat16),
            pltpu.VMEM((2, 2, Q, D), jnp.bfloat16),
            pltpu.VMEM((2, 2, Q, D), jnp.bfloat16),
            pltpu.SemaphoreType.DMA((8,)),
            pltpu.SemaphoreType.DMA,
            pltpu.SemaphoreType.DMA((3, 2, 2)),
            pltpu.SemaphoreType.DMA((3, 2, 2)),
        ],
        compiler_params=pltpu.CompilerParams(collective_id=0),
    )(packed, cum, maskf, E)
